# baseline (speedup 1.0000x reference)
import functools

import jax
import jax.numpy as jnp
from jax import lax
from jax.experimental import pallas as pl
from jax.experimental.pallas import tpu as pltpu
from jax.experimental.pallas import tpu_sc as plsc

_VOCAB = 1000000
_H = 64
_N_CMDS = 1024
_CMD_LEN = 20
_OBS_LEN = 200

_NC, _NS = 2, 16
_NW = _NC * _NS
_CMDS_PER_W = _N_CMDS // _NW
_ROWS_PER_W = _CMDS_PER_W * _CMD_LEN
_IDX_CHUNK = 128
_N_CHUNKS = _ROWS_PER_W // _IDX_CHUNK
_OBS_PAD = 256


def _gumbel_const():
  return jax.random.gumbel(jax.random.key(123), (_N_CMDS, 1), jnp.float32)


def _split_indices(idx_v, pidx_v, par_v, n):
  for k in range(n // 16):
    v = idx_v[pl.ds(k * 16, 16)]
    pidx_v[pl.ds(k * 16, 16)] = lax.shift_right_logical(v, 1)
    par_v[pl.ds(k * 16, 16)] = (v & 1).astype(jnp.float32)


def _pool_rows(rows_v, par_v, out_v, c, n_tok):
  def tok_body(t, accs):
    r = c * n_tok + t
    pv = plsc.load_gather(par_v, [jnp.full((16,), r, jnp.int32)])
    sel = pv != 0.0
    return tuple(
        accs[v]
        + jnp.where(
            sel,
            rows_v[r, pl.ds(_H + v * 16, 16)],
            rows_v[r, pl.ds(v * 16, 16)],
        )
        for v in range(4)
    )

  zeros = tuple(jnp.zeros((16,), jnp.float32) for _ in range(4))
  accs = lax.fori_loop(0, n_tok, tok_body, zeros)
  for v in range(4):
    out_v[c, pl.ds(v * 16, 16)] = accs[v]


@functools.lru_cache(maxsize=1)
def _sc_pool_kernel():
  mesh = plsc.VectorSubcoreMesh(
      core_axis_name="c", subcore_axis_name="s",
      num_cores=_NC, num_subcores=_NS,
  )

  @functools.partial(
      pl.kernel,
      out_type=[
          jax.ShapeDtypeStruct((_N_CMDS, _H), jnp.float32),
          jax.ShapeDtypeStruct((1, _H), jnp.float32),
      ],
      mesh=mesh,
      compiler_params=pltpu.CompilerParams(needs_layout_passes=False),
      scratch_types=[
          pltpu.VMEM((_N_CHUNKS, _IDX_CHUNK), jnp.int32),
          pltpu.VMEM((_N_CHUNKS, _IDX_CHUNK), jnp.int32),
          pltpu.VMEM((_ROWS_PER_W,), jnp.float32),
          pltpu.VMEM((_ROWS_PER_W, 2 * _H), jnp.float32),
          pltpu.VMEM((_CMDS_PER_W, _H), jnp.float32),
          pltpu.VMEM((2, _IDX_CHUNK), jnp.int32),
          pltpu.VMEM((2, _IDX_CHUNK), jnp.int32),
          pltpu.VMEM((_OBS_PAD,), jnp.float32),
          pltpu.VMEM((_OBS_PAD, 2 * _H), jnp.float32),
          pltpu.VMEM((1, _H), jnp.float32),
          pltpu.SemaphoreType.DMA,
      ],
  )
  def sc_kernel(emb_hbm, cmd_idx_hbm, obs_idx_hbm, cmd_out_hbm, obs_out_hbm,
                idx_v, pidx_v, par_v, rows_v, out_v,
                obs_idx_v, obs_pidx_v, obs_par_v, obs_rows_v, obs_out_v, sem):
    wid = lax.axis_index("s") * _NC + lax.axis_index("c")

    pltpu.sync_copy(cmd_idx_hbm.at[wid], idx_v)
    for j in range(_N_CHUNKS):
      _split_indices(idx_v.at[j], pidx_v.at[j], par_v.at[pl.ds(j * _IDX_CHUNK, _IDX_CHUNK)], _IDX_CHUNK)
    copies = [
        pltpu.async_copy(
            emb_hbm.at[pidx_v.at[j]],
            rows_v.at[pl.ds(j * _IDX_CHUNK, _IDX_CHUNK)],
            sem,
        )
        for j in range(_N_CHUNKS)
    ]
    for cp in copies:
      cp.wait()

    def cmd_body(c, carry):
      _pool_rows(rows_v, par_v, out_v, c, _CMD_LEN)
      return carry

    lax.fori_loop(0, _CMDS_PER_W, cmd_body, 0)
    pltpu.sync_copy(out_v, cmd_out_hbm.at[pl.ds(wid * _CMDS_PER_W, _CMDS_PER_W)])

    @pl.when(wid == 0)
    def _():
      pltpu.sync_copy(obs_idx_hbm, obs_idx_v)
      for j in range(2):
        _split_indices(obs_idx_v.at[j], obs_pidx_v.at[j], obs_par_v.at[pl.ds(j * _IDX_CHUNK, _IDX_CHUNK)], _IDX_CHUNK)
      ocopies = [
          pltpu.async_copy(
              emb_hbm.at[obs_pidx_v.at[j]],
              obs_rows_v.at[pl.ds(j * _IDX_CHUNK, _IDX_CHUNK)],
              sem,
          )
          for j in range(2)
      ]
      for cp in ocopies:
        cp.wait()
      _pool_rows(obs_rows_v, obs_par_v, obs_out_v, 0, _OBS_LEN)
      pltpu.sync_copy(obs_out_v, obs_out_hbm)

  return sc_kernel


def _tc_epilogue(cmd_sums_ref, obs_sum_ref, cw_ref, cb_ref, aws_ref, awc_ref,
                 ab_ref, g_ref, scores_ref, idx_ref, value_ref):
  obs_mean = obs_sum_ref[...] * (1.0 / _OBS_LEN)
  value_ref[...] = (
      jnp.sum(obs_mean * cw_ref[...], axis=1, keepdims=True) + cb_ref[...]
  )
  s_state = jnp.sum(obs_mean * aws_ref[...], axis=1, keepdims=True) + ab_ref[...]
  cmd_mean = cmd_sums_ref[...] * (1.0 / _CMD_LEN)
  scores = jnp.sum(cmd_mean * awc_ref[...], axis=1, keepdims=True) + s_state
  scores_ref[...] = scores
  z = scores + g_ref[...]
  m = jnp.max(z)
  iota = lax.broadcasted_iota(jnp.int32, (_N_CMDS, 1), 0)
  idx_ref[...] = jnp.min(
      jnp.where(z == m, iota, jnp.int32(2**30)), axis=0, keepdims=True
  )


def kernel(obs, commands, emb_table, critic_w, critic_b, att_w, att_b):
  emb_packed = emb_table.reshape(_VOCAB // 2, 2 * _H)
  cmd_idx = commands.reshape(_NW, _N_CHUNKS, _IDX_CHUNK)
  obs_idx = jnp.concatenate(
      [obs, jnp.zeros((_OBS_PAD - _OBS_LEN,), jnp.int32)]
  ).reshape(2, _IDX_CHUNK)

  cmd_sums, obs_sum = _sc_pool_kernel()(emb_packed, cmd_idx, obs_idx)

  scores2d, idx2d, value = pl.pallas_call(
      _tc_epilogue,
      out_shape=[
          jax.ShapeDtypeStruct((_N_CMDS, 1), jnp.float32),
          jax.ShapeDtypeStruct((1, 1), jnp.int32),
          jax.ShapeDtypeStruct((1, 1), jnp.float32),
      ],
  )(
      cmd_sums,
      obs_sum,
      critic_w.reshape(1, _H),
      critic_b.reshape(1, 1),
      att_w[:_H].reshape(1, _H),
      att_w[_H:].reshape(1, _H),
      att_b.reshape(1, 1),
      _gumbel_const(),
  )
  return scores2d[:, 0], idx2d[0, 0], value


# speedup vs baseline: 2.1666x; 2.1666x over previous
import functools

import jax
import jax.numpy as jnp
from jax import lax
from jax.experimental import pallas as pl
from jax.experimental.pallas import tpu as pltpu
from jax.experimental.pallas import tpu_sc as plsc

_VOCAB = 1000000
_H = 64
_N_CMDS = 1024
_CMD_LEN = 20
_OBS_LEN = 200

_NC, _NS = 2, 16
_NW = _NC * _NS
_CMDS_PER_W = _N_CMDS // _NW
_ROWS_PER_W = _CMDS_PER_W * _CMD_LEN
_IDX_CHUNK = 128
_N_IDX_ROWS = _ROWS_PER_W // _IDX_CHUNK
_CHUNK = 80
_N_CHUNKS = _ROWS_PER_W // _CHUNK
_OBS_PAD = 256
_N_TILES = _VOCAB // 8


def _gumbel_const():
  return jax.random.gumbel(jax.random.key(123), (_N_CMDS, 1), jnp.float32)


def _extract16(piece, lanes):
  return [jnp.max(jnp.where(lanes == l, piece, jnp.int32(0)))
          for l in range(16)]


@functools.lru_cache(maxsize=1)
def _sc_pool_kernel():
  mesh = plsc.VectorSubcoreMesh(
      core_axis_name="c", subcore_axis_name="s",
      num_cores=_NC, num_subcores=_NS,
  )

  @functools.partial(
      pl.kernel,
      out_type=[
          jax.ShapeDtypeStruct((_N_CMDS, _H), jnp.float32),
          jax.ShapeDtypeStruct((1, _H), jnp.float32),
      ],
      mesh=mesh,
      compiler_params=pltpu.CompilerParams(needs_layout_passes=False),
      scratch_types=[
          pltpu.VMEM((_N_IDX_ROWS, _IDX_CHUNK), jnp.int32),
          pltpu.VMEM((_N_CHUNKS, _CHUNK), jnp.int32),
          pltpu.VMEM((2 * _CMD_LEN, 8, _H), jnp.float32),
          pltpu.VMEM((_CMDS_PER_W, _H), jnp.float32),
          pltpu.VMEM((2, _IDX_CHUNK), jnp.int32),
          pltpu.VMEM((1, _H), jnp.float32),
          pltpu.SemaphoreType.DMA,
      ],
  )
  def sc_kernel(emb_hbm, cmd_idx_hbm, obs_idx_hbm, cmd_out_hbm, obs_out_hbm,
                idx_v, idxr_v, rows_v, out_v, obs_idx_v, obs_out_v, sem):
    wid = lax.axis_index("s") * _NC + lax.axis_index("c")
    lanes = jax.lax.iota(jnp.int32, 16)

    pltpu.sync_copy(cmd_idx_hbm.at[wid], idx_v)
    for i in range(_ROWS_PER_W // 16):
      idxr_v[(16 * i) // _CHUNK, pl.ds((16 * i) % _CHUNK, 16)] = (
          idx_v[i // 8, pl.ds((i % 8) * 16, 16)]
      )

    def fire(toks, m, buf):
      return [
          pltpu.async_copy(
              emb_hbm.at[lax.shift_right_logical(toks[_CMD_LEN * m + o], 3)],
              rows_v.at[buf * _CMD_LEN + o],
              sem,
          )
          for o in range(_CMD_LEN)
      ]

    def pool(toks, m, buf, accs):
      for o in range(_CMD_LEN):
        row = toks[_CMD_LEN * m + o] & 7
        slot = buf * _CMD_LEN + o
        accs = tuple(
            accs[v] + rows_v[slot, row, pl.ds(v * 16, 16)] for v in range(4)
        )
      return accs

    def chunk_body(c, carry):
      toks = []
      for g in range(_CHUNK // 16):
        toks += _extract16(idxr_v[c, pl.ds(16 * g, 16)], lanes)
      cps = fire(toks, 0, 0)
      for m in range(4):
        nxt = fire(toks, m + 1, (m + 1) % 2) if m < 3 else []
        for cp in cps:
          cp.wait()
        accs = pool(toks, m, m % 2,
                    tuple(jnp.zeros((16,), jnp.float32) for _ in range(4)))
        for v in range(4):
          out_v[c * 4 + m, pl.ds(v * 16, 16)] = accs[v]
        cps = nxt
      return carry

    lax.fori_loop(0, _N_CHUNKS, chunk_body, 0)
    pltpu.sync_copy(out_v, cmd_out_hbm.at[pl.ds(wid * _CMDS_PER_W, _CMDS_PER_W)])

    @pl.when(wid == 0)
    def _():
      pltpu.sync_copy(obs_idx_hbm, obs_idx_v)
      toks = []
      for p in range(_OBS_PAD // 16):
        toks += _extract16(obs_idx_v[p // 8, pl.ds((p % 8) * 16, 16)], lanes)
      accs = tuple(jnp.zeros((16,), jnp.float32) for _ in range(4))
      for r in range(5):
        cps = [
            pltpu.async_copy(
                emb_hbm.at[lax.shift_right_logical(toks[40 * r + o], 3)],
                rows_v.at[o],
                sem,
            )
            for o in range(40)
        ]
        for cp in cps:
          cp.wait()
        for o in range(40):
          row = toks[40 * r + o] & 7
          accs = tuple(
              accs[v] + rows_v[o, row, pl.ds(v * 16, 16)] for v in range(4)
          )
      for v in range(4):
        obs_out_v[0, pl.ds(v * 16, 16)] = accs[v]
      pltpu.sync_copy(obs_out_v, obs_out_hbm)

  return sc_kernel


def _tc_epilogue(cmd_sums_ref, obs_sum_ref, cw_ref, cb_ref, aws_ref, awc_ref,
                 ab_ref, g_ref, scores_ref, idx_ref, value_ref):
  obs_mean = obs_sum_ref[...] * (1.0 / _OBS_LEN)
  value_ref[...] = (
      jnp.sum(obs_mean * cw_ref[...], axis=1, keepdims=True) + cb_ref[...]
  )
  s_state = jnp.sum(obs_mean * aws_ref[...], axis=1, keepdims=True) + ab_ref[...]
  cmd_mean = cmd_sums_ref[...] * (1.0 / _CMD_LEN)
  scores = jnp.sum(cmd_mean * awc_ref[...], axis=1, keepdims=True) + s_state
  scores_ref[...] = scores
  z = scores + g_ref[...]
  m = jnp.max(z)
  iota = lax.broadcasted_iota(jnp.int32, (_N_CMDS, 1), 0)
  idx_ref[...] = jnp.min(
      jnp.where(z == m, iota, jnp.int32(2**30)), axis=0, keepdims=True
  )


def kernel(obs, commands, emb_table, critic_w, critic_b, att_w, att_b):
  emb_tiles = emb_table.reshape(_N_TILES, 8, _H)
  cmd_idx = commands.reshape(_NW, _N_IDX_ROWS, _IDX_CHUNK)
  obs_idx = jnp.concatenate(
      [obs, jnp.zeros((_OBS_PAD - _OBS_LEN,), jnp.int32)]
  ).reshape(2, _IDX_CHUNK)

  cmd_sums, obs_sum = _sc_pool_kernel()(emb_tiles, cmd_idx, obs_idx)

  scores2d, idx2d, value = pl.pallas_call(
      _tc_epilogue,
      out_shape=[
          jax.ShapeDtypeStruct((_N_CMDS, 1), jnp.float32),
          jax.ShapeDtypeStruct((1, 1), jnp.int32),
          jax.ShapeDtypeStruct((1, 1), jnp.float32),
      ],
  )(
      cmd_sums,
      obs_sum,
      critic_w.reshape(1, _H),
      critic_b.reshape(1, 1),
      att_w[:_H].reshape(1, _H),
      att_w[_H:].reshape(1, _H),
      att_b.reshape(1, 1),
      _gumbel_const(),
  )
  return scores2d[:, 0], idx2d[0, 0], value


# speedup vs baseline: 3.2481x; 1.4992x over previous
import functools

import jax
import jax.numpy as jnp
from jax import lax
from jax.experimental import pallas as pl
from jax.experimental.pallas import tpu as pltpu
from jax.experimental.pallas import tpu_sc as plsc

_VOCAB = 1000000
_H = 64
_N_CMDS = 1024
_CMD_LEN = 20
_OBS_LEN = 200

_NC, _NS = 2, 16
_NW = _NC * _NS
_CMDS_PER_W = _N_CMDS // _NW
_ROWS_PER_W = _CMDS_PER_W * _CMD_LEN
_IDX_CHUNK = 128
_N_CHUNKS = _ROWS_PER_W // _IDX_CHUNK
_OBS_PAD = 256
_P_ROWS = _VOCAB // 16
_VB = 8192
_GRID_A = -(-_VOCAB // _VB)


def _gumbel_const():
  return jax.random.gumbel(
      jax.random.key(123), (_N_CMDS,), jnp.float32
  ).reshape(_NW, _CMDS_PER_W)


def _matvec_kernel(embT_ref, w_ref, pc_ref, ps_ref, pv_ref):
  p = lax.dot_general(
      w_ref[...], embT_ref[...], (((1,), (0,)), ((), ())),
      preferred_element_type=jnp.float32,
      precision=lax.Precision.HIGHEST,
  )
  pc_ref[...] = p[0]
  ps_ref[...] = p[1]
  pv_ref[...] = p[2]


def _projections(embT, w8):
  return pl.pallas_call(
      _matvec_kernel,
      grid=(_GRID_A,),
      in_specs=[
          pl.BlockSpec((_H, _VB), lambda i: (0, i)),
          pl.BlockSpec((8, _H), lambda i: (0, 0)),
      ],
      out_specs=[
          pl.BlockSpec((_VB,), lambda i: (i,)),
          pl.BlockSpec((_VB,), lambda i: (i,)),
          pl.BlockSpec((_VB,), lambda i: (i,)),
      ],
      out_shape=[jax.ShapeDtypeStruct((_VOCAB,), jnp.float32)] * 3,
  )(embT, w8)


@functools.lru_cache(maxsize=1)
def _sc_pool_kernel():
  mesh = plsc.VectorSubcoreMesh(
      core_axis_name="c", subcore_axis_name="s",
      num_cores=_NC, num_subcores=_NS,
  )

  @functools.partial(
      pl.kernel,
      out_type=[
          jax.ShapeDtypeStruct((_NW, 1, _CMDS_PER_W), jnp.float32),
          jax.ShapeDtypeStruct((2, 1, 16), jnp.float32),
      ],
      mesh=mesh,
      compiler_params=pltpu.CompilerParams(
          use_tc_tiling_on_sc=False, needs_layout_passes=False
      ),
      scratch_types=[
          pltpu.VMEM((_N_CHUNKS, _IDX_CHUNK), jnp.int32),
          pltpu.VMEM((_N_CHUNKS, _IDX_CHUNK), jnp.int32),
          pltpu.VMEM((_ROWS_PER_W, 16), jnp.float32),
          pltpu.VMEM((1, 2 * 16), jnp.float32),
          pltpu.VMEM((2, _IDX_CHUNK), jnp.int32),
          pltpu.VMEM((_OBS_PAD, 16), jnp.float32),
          pltpu.VMEM((1, 16), jnp.float32),
          pltpu.SemaphoreType.DMA,
      ],
  )
  def sc_kernel(pc_hbm, ps_hbm, pv_hbm, cmd_idx_hbm, obs_idx_hbm,
                cmd_out_hbm, obs_out_hbm,
                idx_v, row_v, rows_v, out_v, oidx_v, orows_v, oacc_v, sem):
    wid = lax.axis_index("s") * _NC + lax.axis_index("c")
    lanes = lax.iota(jnp.int32, 16)

    pltpu.sync_copy(cmd_idx_hbm.at[wid], idx_v)
    for j in range(_N_CHUNKS):
      for g in range(8):
        row_v[j, pl.ds(g * 16, 16)] = lax.shift_right_logical(
            idx_v[j, pl.ds(g * 16, 16)], 4
        )
    cps = [
        pltpu.async_copy(
            pc_hbm.at[row_v.at[j]],
            rows_v.at[pl.ds(j * _IDX_CHUNK, _IDX_CHUNK)],
            sem,
        )
        for j in range(_N_CHUNKS)
    ]
    for cp in cps:
      cp.wait()

    accs = [jnp.zeros((16,), jnp.float32) for _ in range(2)]
    for g in range(_ROWS_PER_W // 16):
      j, o = g // 8, (g % 8) * 16
      sel = idx_v[j, pl.ds(o, 16)] & 15
      val = plsc.load_gather(rows_v, [lanes + 16 * g, sel])
      accs[g % 2] = accs[g % 2] + val
    out_v[0, pl.ds(0, 16)] = accs[0]
    out_v[0, pl.ds(16, 16)] = accs[1]
    pltpu.sync_copy(out_v, cmd_out_hbm.at[wid])

    def obs_pool(tbl_hbm, out_row):
      pltpu.sync_copy(obs_idx_hbm, oidx_v)
      for j in range(2):
        for g in range(8):
          row_v[j, pl.ds(g * 16, 16)] = lax.shift_right_logical(
              oidx_v[j, pl.ds(g * 16, 16)], 4
          )
      ocps = [
          pltpu.async_copy(
              tbl_hbm.at[row_v.at[j]],
              orows_v.at[pl.ds(j * _IDX_CHUNK, _IDX_CHUNK)],
              sem,
          )
          for j in range(2)
      ]
      for cp in ocps:
        cp.wait()
      oacc = jnp.zeros((16,), jnp.float32)
      for g in range(13):
        j, o = g // 8, (g % 8) * 16
        sel = oidx_v[j, pl.ds(o, 16)] & 15
        val = plsc.load_gather(orows_v, [lanes + 16 * g, sel])
        if g == 12:
          val = jnp.where(lanes < _OBS_LEN - 12 * 16, val, 0.0)
        oacc = oacc + val
      oacc_v[0, pl.ds(0, 16)] = oacc
      pltpu.sync_copy(oacc_v, obs_out_hbm.at[out_row])

    @pl.when(wid == 31)
    def _():
      obs_pool(ps_hbm, 0)

    @pl.when(wid == 30)
    def _():
      obs_pool(pv_hbm, 1)

  return sc_kernel


def _tc_epilogue(cmd_ref, obs_ref, cb_ref, ab_ref, g_ref,
                 scores_ref, idx_ref, value_ref):
  obs = obs_ref[...]
  state_mean = jnp.sum(obs[0:1, :]) * (1.0 / _OBS_LEN)
  value_ref[...] = (
      jnp.sum(obs[1:2, :], keepdims=True) * (1.0 / _OBS_LEN) + cb_ref[...]
  )
  scores = cmd_ref[...] * (1.0 / _CMD_LEN) + state_mean + ab_ref[0, 0]
  scores_ref[...] = scores
  z = scores + g_ref[...]
  m = jnp.max(z)
  iw = lax.broadcasted_iota(jnp.int32, (_NW, _CMDS_PER_W), 0)
  ic = lax.broadcasted_iota(jnp.int32, (_NW, _CMDS_PER_W), 1)
  idx_ref[...] = jnp.min(
      jnp.where(z == m, iw * _CMDS_PER_W + ic, jnp.int32(2**30)),
      keepdims=True,
  )


def kernel(obs, commands, emb_table, critic_w, critic_b, att_w, att_b):
  embT = emb_table.T
  w8 = jnp.zeros((8, _H), jnp.float32)
  w8 = w8.at[0].set(att_w[_H:, 0]).at[1].set(att_w[:_H, 0])
  w8 = w8.at[2].set(critic_w[:, 0])
  p_cmd, p_state, p_crit = _projections(embT, w8)

  cmd_idx = (
      commands.reshape(_NW, _CMDS_PER_W, _CMD_LEN)
      .transpose(0, 2, 1)
      .reshape(_NW, _N_CHUNKS, _IDX_CHUNK)
  )
  obs_idx = jnp.concatenate(
      [obs, jnp.zeros((_OBS_PAD - _OBS_LEN,), jnp.int32)]
  ).reshape(2, _IDX_CHUNK)

  cmd_sums, obs_accs = _sc_pool_kernel()(
      p_cmd.reshape(_P_ROWS, 16),
      p_state.reshape(_P_ROWS, 16),
      p_crit.reshape(_P_ROWS, 16),
      cmd_idx,
      obs_idx,
  )

  scores2d, idx2d, value = pl.pallas_call(
      _tc_epilogue,
      out_shape=[
          jax.ShapeDtypeStruct((_NW, _CMDS_PER_W), jnp.float32),
          jax.ShapeDtypeStruct((1, 1), jnp.int32),
          jax.ShapeDtypeStruct((1, 1), jnp.float32),
      ],
  )(
      cmd_sums.reshape(_NW, _CMDS_PER_W),
      obs_accs.reshape(2, 16),
      critic_b.reshape(1, 1),
      att_b.reshape(1, 1),
      _gumbel_const(),
  )
  return scores2d.reshape(_N_CMDS), idx2d[0, 0], value
